# baseline (device time: 96169 ns/iter reference)
import os

import jax
import jax.numpy as jnp
from jax import lax
from jax.experimental import pallas as pl
from jax.experimental.pallas import tpu as pltpu

_COMM = os.environ.get("KERNEL_NO_COMM") != "1"

N_DEV = 4
SQ = 2048
D_MODEL = 1024
H_LOC = 8
DH = 128
WIN = 128
SCALE = 0.08838834764831843
QB = 256
KB = 512
NSC = SQ // QB


def _body(x_ref, wq_ref, k_ref, v_ref, wo_ref, out_ref,
          q_ref, ctx_ref, wq_bf, wo_bf, sbuf, rbuf,
          ssems_rs, rsems_rs, ssems_ag, rsems_ag):
    my = lax.axis_index("i")

    wq_bf[...] = (wq_ref[...] * SCALE).astype(jnp.bfloat16)
    wo_bf[...] = wo_ref[...].astype(jnp.bfloat16)

    if _COMM:
        barrier = pltpu.get_barrier_semaphore()
        for d in (1, 2, 3):
            pl.semaphore_signal(
                barrier, inc=1, device_id=(lax.rem(my + d, N_DEV),),
                device_id_type=pl.DeviceIdType.MESH,
            )
        pl.semaphore_wait(barrier, 3)

    sends = []
    for t in range(NSC):
        s_t = 4 * (t // 4) + lax.rem(my + 1 + t, N_DEV)
        r0 = pl.multiple_of(s_t * QB, QB)

        q_ref[...] = jnp.dot(
            x_ref[pl.ds(r0, QB), :].astype(jnp.bfloat16), wq_bf[...],
            preferred_element_type=jnp.float32,
        ).astype(jnp.bfloat16)

        skey = pl.multiple_of(jnp.clip(r0 - WIN, 0, SQ - KB), WIN)
        rows = lax.broadcasted_iota(jnp.int32, (QB, KB), 0) + r0
        cols = lax.broadcasted_iota(jnp.int32, (QB, KB), 1) + skey
        keep = jnp.abs(rows - cols) <= WIN
        for h in range(H_LOC):
            hc = h * DH
            qh = q_ref[:, hc:hc + DH]
            kh = k_ref[pl.ds(skey, KB), hc:hc + DH]
            vh = v_ref[pl.ds(skey, KB), hc:hc + DH]
            sc = lax.dot_general(
                qh, kh, (((1,), (1,)), ((), ())),
                preferred_element_type=jnp.float32,
            )
            w = jnp.exp(jnp.where(keep, sc, -1e9))
            denom = jnp.sum(w, axis=1, keepdims=True)
            ctx = jnp.dot(
                w.astype(jnp.bfloat16), vh,
                preferred_element_type=jnp.float32,
            )
            ctx_ref[:, hc:hc + DH] = (ctx / denom).astype(jnp.bfloat16)

        p = jnp.dot(
            ctx_ref[...], wo_bf[...],
            preferred_element_type=jnp.float32,
        )

        if t not in (3, 7):
            idx = t if t < 3 else t - 1
            d_recv = (4 - (1 + t) % 4) % 4
            half_s = 1 - t // 4
            sbuf[idx] = p.astype(jnp.bfloat16)
            if _COMM:
                owner = lax.rem(s_t, N_DEV)
                rdma = pltpu.make_async_remote_copy(
                    src_ref=sbuf.at[idx],
                    dst_ref=rbuf.at[d_recv - 1, half_s],
                    send_sem=ssems_rs.at[idx],
                    recv_sem=rsems_rs.at[d_recv - 1, half_s],
                    device_id=(owner,),
                    device_id_type=pl.DeviceIdType.MESH,
                )
                rdma.start()
                sends.append(rdma)
        else:
            half = 1 if t == 3 else 0
            if _COMM:
                for dd in (1, 2, 3):
                    recv = pltpu.make_async_remote_copy(
                        src_ref=sbuf.at[0], dst_ref=rbuf.at[dd - 1, half],
                        send_sem=ssems_rs.at[0],
                        recv_sem=rsems_rs.at[dd - 1, half],
                        device_id=(my,), device_id_type=pl.DeviceIdType.MESH,
                    )
                    recv.wait_recv()
                    p = p + rbuf[dd - 1, half].astype(jnp.float32)
            out_ref[pl.ds(r0, QB), :] = p.astype(jnp.bfloat16)
            if _COMM:
                for j, d in enumerate((2, 1, 3)):
                    tgt = lax.rem(my + d, N_DEV)
                    rdma = pltpu.make_async_remote_copy(
                        src_ref=out_ref.at[pl.ds(r0, QB), :],
                        dst_ref=out_ref.at[pl.ds(r0, QB), :],
                        send_sem=ssems_ag.at[3 * half + j],
                        recv_sem=rsems_ag.at[my, half],
                        device_id=(tgt,),
                        device_id_type=pl.DeviceIdType.MESH,
                    )
                    rdma.start()
                    sends.append(rdma)

    if not _COMM:
        return

    for d in (1, 3, 2):
        p_id = lax.rem(my + d, N_DEV)
        for half in range(2):
            pr0 = pl.multiple_of(
                p_id * QB + (1 - half) * (N_DEV * QB), QB
            )
            recv = pltpu.make_async_remote_copy(
                src_ref=out_ref.at[pl.ds(pr0, QB), :],
                dst_ref=out_ref.at[pl.ds(pr0, QB), :],
                send_sem=ssems_ag.at[0], recv_sem=rsems_ag.at[p_id, half],
                device_id=(p_id,), device_id_type=pl.DeviceIdType.MESH,
            )
            recv.wait_recv()
    for rdma in sends:
        rdma.wait_send()


def kernel(x, Wq, K_ext, V_ext, Wo):
    i = lax.axis_index("i")
    xb = x.reshape(SQ, D_MODEL)
    k = lax.dynamic_slice(
        K_ext, (0, 0, i * H_LOC, 0), (1, SQ, H_LOC, DH)
    ).reshape(SQ, H_LOC * DH).astype(jnp.bfloat16)
    v = lax.dynamic_slice(
        V_ext, (0, 0, i * H_LOC, 0), (1, SQ, H_LOC, DH)
    ).reshape(SQ, H_LOC * DH).astype(jnp.bfloat16)

    out = pl.pallas_call(
        _body,
        out_shape=jax.ShapeDtypeStruct((SQ, D_MODEL), jnp.bfloat16),
        in_specs=[pl.BlockSpec(memory_space=pltpu.VMEM)] * 5,
        out_specs=pl.BlockSpec(memory_space=pltpu.VMEM),
        scratch_shapes=[
            pltpu.VMEM((QB, D_MODEL), jnp.bfloat16),
            pltpu.VMEM((QB, D_MODEL), jnp.bfloat16),
            pltpu.VMEM((D_MODEL, D_MODEL), jnp.bfloat16),
            pltpu.VMEM((D_MODEL, D_MODEL), jnp.bfloat16),
            pltpu.VMEM((6, QB, D_MODEL), jnp.bfloat16),
            pltpu.VMEM((3, 2, QB, D_MODEL), jnp.bfloat16),
            pltpu.SemaphoreType.DMA((6,)),
            pltpu.SemaphoreType.DMA((3, 2)),
            pltpu.SemaphoreType.DMA((6,)),
            pltpu.SemaphoreType.DMA((N_DEV, 2)),
        ],
        compiler_params=pltpu.CompilerParams(
            collective_id=0 if _COMM else None,
            vmem_limit_bytes=96 * 1024 * 1024,
        ),
    )(xb, Wq, k, v, Wo)
    return out.reshape(1, SQ, D_MODEL)


# device time: 94529 ns/iter; 1.0173x vs baseline; 1.0173x over previous
import os

import jax
import jax.numpy as jnp
from jax import lax
from jax.experimental import pallas as pl
from jax.experimental.pallas import tpu as pltpu

_COMM = os.environ.get("KERNEL_NO_COMM") != "1"

N_DEV = 4
SQ = 2048
D_MODEL = 1024
H_LOC = 8
DH = 128
WIN = 128
SCALE = 0.08838834764831843
CHUNK = SQ // N_DEV
QB = 256
KB = 512
NSUB = CHUNK // QB


def _body(x_ref, wq_ref, k_ref, v_ref, wo_ref, out_ref,
          q_ref, ctx_ref, acc_ref, wq_bf, wo_bf, sbuf, rbuf_rs,
          ssems_rs, rsems_rs, ssems_ag, rsems_ag):
    my = lax.axis_index("i")

    wq_bf[...] = (wq_ref[...] * SCALE).astype(jnp.bfloat16)
    wo_bf[...] = wo_ref[...].astype(jnp.bfloat16)

    if _COMM:
        barrier = pltpu.get_barrier_semaphore()
        for d in (1, 2, 3):
            pl.semaphore_signal(
                barrier, inc=1, device_id=(lax.rem(my + d, N_DEV),),
                device_id_type=pl.DeviceIdType.MESH,
            )
        pl.semaphore_wait(barrier, 3)

    rs_sends = []
    slot_of_d = {2: 0, 1: 1, 3: 2}
    for d in (2, 1, 3, 0):
        c = lax.rem(my + d, N_DEV)
        r0 = pl.multiple_of(c * CHUNK, CHUNK)

        q_ref[...] = jnp.dot(
            x_ref[pl.ds(r0, CHUNK), :].astype(jnp.bfloat16), wq_bf[...],
            preferred_element_type=jnp.float32,
        ).astype(jnp.bfloat16)

        for b in range(NSUB):
            row0 = r0 + b * QB
            s = pl.multiple_of(jnp.clip(row0 - WIN, 0, SQ - KB), WIN)
            rows = lax.broadcasted_iota(jnp.int32, (QB, KB), 0) + row0
            cols = lax.broadcasted_iota(jnp.int32, (QB, KB), 1) + s
            keep = jnp.abs(rows - cols) <= WIN

            for h in range(H_LOC):
                hc = h * DH
                qh = q_ref[pl.ds(b * QB, QB), hc:hc + DH]
                kh = k_ref[pl.ds(s, KB), hc:hc + DH]
                vh = v_ref[pl.ds(s, KB), hc:hc + DH]
                sc = lax.dot_general(
                    qh, kh, (((1,), (1,)), ((), ())),
                    preferred_element_type=jnp.float32,
                )
                w = jnp.exp(jnp.where(keep, sc, -1e9))
                denom = jnp.sum(w, axis=1, keepdims=True)
                ctx = jnp.dot(
                    w.astype(jnp.bfloat16), vh,
                    preferred_element_type=jnp.float32,
                )
                ctx_ref[b * QB:(b + 1) * QB, hc:hc + DH] = (
                    (ctx / denom).astype(jnp.bfloat16)
                )

        p_c = jnp.dot(
            ctx_ref[...], wo_bf[...],
            preferred_element_type=jnp.float32,
        )
        if d == 0:
            acc_ref[...] = p_c
        else:
            slot = slot_of_d[d]
            sbuf[slot] = p_c.astype(jnp.bfloat16)
            if _COMM:
                rdma = pltpu.make_async_remote_copy(
                    src_ref=sbuf.at[slot],
                    dst_ref=rbuf_rs.at[my],
                    send_sem=ssems_rs.at[slot],
                    recv_sem=rsems_rs.at[my],
                    device_id=(c,),
                    device_id_type=pl.DeviceIdType.MESH,
                )
                rdma.start()
                rs_sends.append(rdma)

    if not _COMM:
        out_ref[pl.ds(my * CHUNK, CHUNK), :] = acc_ref[...].astype(jnp.bfloat16)
        return

    for d in (1, 3, 2):
        s = lax.rem(my + d, N_DEV)
        recv = pltpu.make_async_remote_copy(
            src_ref=sbuf.at[0], dst_ref=rbuf_rs.at[s],
            send_sem=ssems_rs.at[0], recv_sem=rsems_rs.at[s],
            device_id=(s,), device_id_type=pl.DeviceIdType.MESH,
        )
        recv.wait_recv()
    for rdma in rs_sends:
        rdma.wait_send()

    ag_sends = []
    for half in range(2):
        hr = half * QB
        my_h0 = pl.multiple_of(my * CHUNK + hr, QB)
        part = acc_ref[hr:hr + QB, :]
        for d in (1, 3, 2):
            s = lax.rem(my + d, N_DEV)
            part = part + rbuf_rs[s, hr:hr + QB, :].astype(jnp.float32)
        out_ref[pl.ds(my_h0, QB), :] = part.astype(jnp.bfloat16)
        for d in (2, 1, 3):
            j = lax.rem(my + d, N_DEV)
            rdma = pltpu.make_async_remote_copy(
                src_ref=out_ref.at[pl.ds(my_h0, QB), :],
                dst_ref=out_ref.at[pl.ds(my_h0, QB), :],
                send_sem=ssems_ag.at[3 * half + d - 1],
                recv_sem=rsems_ag.at[my, half],
                device_id=(j,),
                device_id_type=pl.DeviceIdType.MESH,
            )
            rdma.start()
            ag_sends.append(rdma)

    for d in (1, 3, 2):
        s = lax.rem(my + d, N_DEV)
        for half in range(2):
            s_h0 = pl.multiple_of(s * CHUNK + half * QB, QB)
            recv = pltpu.make_async_remote_copy(
                src_ref=out_ref.at[pl.ds(s_h0, QB), :],
                dst_ref=out_ref.at[pl.ds(s_h0, QB), :],
                send_sem=ssems_ag.at[0], recv_sem=rsems_ag.at[s, half],
                device_id=(s,), device_id_type=pl.DeviceIdType.MESH,
            )
            recv.wait_recv()
    for rdma in ag_sends:
        rdma.wait_send()


def kernel(x, Wq, K_ext, V_ext, Wo):
    i = lax.axis_index("i")
    xb = x.reshape(SQ, D_MODEL)
    k = lax.dynamic_slice(
        K_ext, (0, 0, i * H_LOC, 0), (1, SQ, H_LOC, DH)
    ).reshape(SQ, H_LOC * DH).astype(jnp.bfloat16)
    v = lax.dynamic_slice(
        V_ext, (0, 0, i * H_LOC, 0), (1, SQ, H_LOC, DH)
    ).reshape(SQ, H_LOC * DH).astype(jnp.bfloat16)

    out = pl.pallas_call(
        _body,
        out_shape=jax.ShapeDtypeStruct((SQ, D_MODEL), jnp.bfloat16),
        in_specs=[pl.BlockSpec(memory_space=pltpu.VMEM)] * 5,
        out_specs=pl.BlockSpec(memory_space=pltpu.VMEM),
        scratch_shapes=[
            pltpu.VMEM((CHUNK, D_MODEL), jnp.bfloat16),
            pltpu.VMEM((CHUNK, D_MODEL), jnp.bfloat16),
            pltpu.VMEM((CHUNK, D_MODEL), jnp.float32),
            pltpu.VMEM((D_MODEL, D_MODEL), jnp.bfloat16),
            pltpu.VMEM((D_MODEL, D_MODEL), jnp.bfloat16),
            pltpu.VMEM((3, CHUNK, D_MODEL), jnp.bfloat16),
            pltpu.VMEM((N_DEV, CHUNK, D_MODEL), jnp.bfloat16),
            pltpu.SemaphoreType.DMA((3,)),
            pltpu.SemaphoreType.DMA((N_DEV,)),
            pltpu.SemaphoreType.DMA((6,)),
            pltpu.SemaphoreType.DMA((N_DEV, 2)),
        ],
        compiler_params=pltpu.CompilerParams(
            collective_id=0 if _COMM else None,
            vmem_limit_bytes=96 * 1024 * 1024,
        ),
    )(xb, Wq, k, v, Wo)
    return out.reshape(1, SQ, D_MODEL)
